# Initial kernel scaffold; baseline (speedup 1.0000x reference)
import jax
import jax.numpy as jnp
from jax.experimental import pallas as pl


def kernel(pred, target, batch, pos):
    raise NotImplementedError("write your pallas kernel here")


# speedup vs baseline: 2.5276x; 2.5276x over previous
import functools

import jax
import jax.numpy as jnp
from jax import lax
from jax.experimental import pallas as pl
from jax.experimental.pallas import tpu as pltpu
from jax.experimental.pallas import tpu_sc as plsc

N = 1600000
S = 1024
NW = 32
PW = N // NW
CHUNK = 10000
NCHUNK = PW // CHUNK
NVEC = CHUNK // 16
NQ = 9

_mesh = plsc.VectorSubcoreMesh(core_axis_name="c", subcore_axis_name="s")


@functools.partial(
    pl.kernel,
    mesh=_mesh,
    out_type=jax.ShapeDtypeStruct((NW * NQ * S,), jnp.float32),
    compiler_params=pltpu.CompilerParams(needs_layout_passes=False),
    scratch_types=[
        pltpu.VMEM((CHUNK,), jnp.float32),
        pltpu.VMEM((CHUNK,), jnp.float32),
        pltpu.VMEM((CHUNK,), jnp.int32),
        pltpu.VMEM((3 * CHUNK,), jnp.float32),
    ] + [pltpu.VMEM((S,), jnp.float32) for _ in range(NQ)],
)
def _seg_partials(pred_hbm, targ_hbm, batch_hbm, posf_hbm, out_hbm,
                  pred_v, targ_v, batch_v, pos_v, *accs):
    wid = lax.axis_index("s") * 2 + lax.axis_index("c")
    base_w = wid * PW

    zero16 = jnp.zeros((16,), jnp.float32)

    def zero_body(i, carry):
        for a in accs:
            a[pl.ds(i * 16, 16)] = zero16
        return carry

    lax.fori_loop(0, S // 16, zero_body, 0)

    lane = lax.iota(jnp.int32, 16)
    shift = jnp.minimum(lane + 1, 15)
    lane_lt15 = lane < 15
    lane15 = lane == 15
    pidx = lane * 3

    def chunk_body(c, carry):
        base = base_w + c * CHUNK
        pltpu.sync_copy(pred_hbm.at[pl.ds(base, CHUNK)], pred_v)
        pltpu.sync_copy(targ_hbm.at[pl.ds(base, CHUNK)], targ_v)
        pltpu.sync_copy(batch_hbm.at[pl.ds(base, CHUNK)], batch_v)
        pltpu.sync_copy(posf_hbm.at[pl.ds(base * 3, 3 * CHUNK)], pos_v)

        def vec_body(i, inner):
            off = i * 16
            b = batch_v[pl.ds(off, 16)]
            p = pred_v[pl.ds(off, 16)]
            t = targ_v[pl.ds(off, 16)]
            gx = pidx + off * 3
            px = plsc.load_gather(pos_v, [gx])
            py = plsc.load_gather(pos_v, [gx + 1])
            pz = plsc.load_gather(pos_v, [gx + 2])
            bnext = plsc.load_gather(batch_v, [off + shift])
            m = (b != bnext) & lane_lt15
            m1 = m | lane15
            vals = (p, t, p * t, px * t, py * t, pz * t, px * p, py * p, pz * p)
            for a, vq in zip(accs, vals):
                cs = plsc.cumsum(vq)
                plsc.addupdate_scatter(a, [b], cs, mask=m1)
                plsc.addupdate_scatter(a, [bnext], -cs, mask=m)
            return inner

        lax.fori_loop(0, NVEC, vec_body, 0)
        return carry

    lax.fori_loop(0, NCHUNK, chunk_body, 0)

    for q, a in enumerate(accs):
        pltpu.sync_copy(a, out_hbm.at[pl.ds((wid * NQ + q) * S, S)])


def _epilogue_body(parts_ref, out_ref):
    acc = jnp.sum(parts_ref[...], axis=0)
    p_sum = acc[0:1]
    t_sum = acc[1:2]
    inter = acc[2:3]
    dice = (2.0 * inter + 1.0) / (p_sum + t_sum + 1.0)
    dice_loss = jnp.sum(1.0 - dice)
    t_center = acc[3:6] / (t_sum + 1e-10)
    p_center = acc[6:9] / (p_sum + 1e-10)
    diff = t_center - p_center
    pcl = jnp.sqrt(jnp.sum(diff * diff))
    total = dice_loss + pcl
    row_i = lax.broadcasted_iota(jnp.int32, (8, 128), 0)
    col_i = lax.broadcasted_iota(jnp.int32, (8, 128), 1)
    vals = jnp.where(col_i == 0, total,
                     jnp.where(col_i == 1, dice_loss,
                               jnp.where(col_i == 2, pcl, 0.0)))
    out_ref[...] = jnp.where(row_i == 0, vals, 0.0)


def kernel(pred, target, batch, pos):
    pred = pred.reshape(-1).astype(jnp.float32)
    target = target.reshape(-1).astype(jnp.float32)
    posf = pos.reshape(-1).astype(jnp.float32)
    batch = batch.astype(jnp.int32)
    parts = _seg_partials(pred, target, batch, posf).reshape(NW, NQ, S)
    out = pl.pallas_call(
        _epilogue_body,
        out_shape=jax.ShapeDtypeStruct((8, 128), jnp.float32),
    )(parts)
    ce = jnp.zeros((), jnp.float32)
    return (out[0, 0], out[0, 1], out[0, 2], ce)


# speedup vs baseline: 27.9267x; 11.0487x over previous
import functools

import jax
import jax.numpy as jnp
from jax import lax
from jax.experimental import pallas as pl
from jax.experimental.pallas import tpu as pltpu
from jax.experimental.pallas import tpu_sc as plsc

N = 1600000
S = 1024
NW = 32
PW = N // NW
CHUNK = 10000
NCHUNK = PW // CHUNK
NVEC = CHUNK // 16
NQ = 9

_mesh = plsc.VectorSubcoreMesh(core_axis_name="c", subcore_axis_name="s")


@functools.partial(
    pl.kernel,
    mesh=_mesh,
    out_type=jax.ShapeDtypeStruct((NW * NQ * S,), jnp.float32),
    compiler_params=pltpu.CompilerParams(needs_layout_passes=False),
    scratch_types=[
        pltpu.VMEM((CHUNK,), jnp.float32),
        pltpu.VMEM((CHUNK,), jnp.float32),
        pltpu.VMEM((CHUNK,), jnp.int32),
        pltpu.VMEM((CHUNK,), jnp.float32),
        pltpu.VMEM((CHUNK,), jnp.float32),
        pltpu.VMEM((CHUNK,), jnp.float32),
    ] + [pltpu.VMEM((S,), jnp.float32) for _ in range(NQ)],
)
def _seg_partials(pred_hbm, targ_hbm, batch_hbm, posf_hbm, out_hbm,
                  pred_v, targ_v, batch_v, px_v, py_v, pz_v, *accs):
    wid = lax.axis_index("s") * 2 + lax.axis_index("c")
    base_w = wid * PW

    zero16 = jnp.zeros((16,), jnp.float32)

    def zero_body(i, carry):
        for a in accs:
            a[pl.ds(i * 16, 16)] = zero16
        return carry

    lax.fori_loop(0, S // 16, zero_body, 0)

    lane = lax.iota(jnp.int32, 16)
    shift = jnp.minimum(lane + 1, 15)
    lane_lt15 = lane < 15
    lane15 = lane == 15

    def chunk_body(c, carry):
        base = base_w + c * CHUNK
        pltpu.sync_copy(pred_hbm.at[pl.ds(base, CHUNK)], pred_v)
        pltpu.sync_copy(targ_hbm.at[pl.ds(base, CHUNK)], targ_v)
        pltpu.sync_copy(batch_hbm.at[pl.ds(base, CHUNK)], batch_v)
        pltpu.sync_copy(posf_hbm.at[pl.ds(base, CHUNK)], px_v)
        pltpu.sync_copy(posf_hbm.at[pl.ds(N + base, CHUNK)], py_v)
        pltpu.sync_copy(posf_hbm.at[pl.ds(2 * N + base, CHUNK)], pz_v)

        def vec_body(i, inner):
            off = i * 16
            b = batch_v[pl.ds(off, 16)]
            p = pred_v[pl.ds(off, 16)]
            t = targ_v[pl.ds(off, 16)]
            px = px_v[pl.ds(off, 16)]
            py = py_v[pl.ds(off, 16)]
            pz = pz_v[pl.ds(off, 16)]
            bnext = plsc.load_gather(batch_v, [off + shift])
            m = (b != bnext) & lane_lt15
            m1 = m | lane15
            vals = (p, t, p * t, px * t, py * t, pz * t, px * p, py * p, pz * p)
            for a, vq in zip(accs, vals):
                cs = plsc.cumsum(vq)
                plsc.addupdate_scatter(a, [b], cs, mask=m1)
                plsc.addupdate_scatter(a, [bnext], -cs, mask=m)
            return inner

        lax.fori_loop(0, NVEC, vec_body, 0)
        return carry

    lax.fori_loop(0, NCHUNK, chunk_body, 0)

    for q, a in enumerate(accs):
        pltpu.sync_copy(a, out_hbm.at[pl.ds((wid * NQ + q) * S, S)])


def _epilogue_body(parts_ref, out_ref):
    acc = jnp.sum(parts_ref[...], axis=0)
    p_sum = acc[0:1]
    t_sum = acc[1:2]
    inter = acc[2:3]
    dice = (2.0 * inter + 1.0) / (p_sum + t_sum + 1.0)
    dice_loss = jnp.sum(1.0 - dice)
    t_center = acc[3:6] / (t_sum + 1e-10)
    p_center = acc[6:9] / (p_sum + 1e-10)
    diff = t_center - p_center
    pcl = jnp.sqrt(jnp.sum(diff * diff))
    total = dice_loss + pcl
    row_i = lax.broadcasted_iota(jnp.int32, (8, 128), 0)
    col_i = lax.broadcasted_iota(jnp.int32, (8, 128), 1)
    vals = jnp.where(col_i == 0, total,
                     jnp.where(col_i == 1, dice_loss,
                               jnp.where(col_i == 2, pcl, 0.0)))
    out_ref[...] = jnp.where(row_i == 0, vals, 0.0)


def kernel(pred, target, batch, pos):
    pred = pred.reshape(-1).astype(jnp.float32)
    target = target.reshape(-1).astype(jnp.float32)
    posf = pos.astype(jnp.float32).T.reshape(-1)
    batch = batch.astype(jnp.int32)
    parts = _seg_partials(pred, target, batch, posf).reshape(NW, NQ, S)
    out = pl.pallas_call(
        _epilogue_body,
        out_shape=jax.ShapeDtypeStruct((8, 128), jnp.float32),
    )(parts)
    ce = jnp.zeros((), jnp.float32)
    return (out[0, 0], out[0, 1], out[0, 2], ce)


# speedup vs baseline: 109.9279x; 3.9363x over previous
import functools

import jax
import jax.numpy as jnp
from jax import lax
from jax.experimental import pallas as pl
from jax.experimental.pallas import tpu as pltpu
from jax.experimental.pallas import tpu_sc as plsc

N = 1600000
S = 1024
NW = 32
BLK = 128
NBLK = N // BLK
CB = 78
CHUNK = CB * BLK
NQ = 9

_mesh = plsc.VectorSubcoreMesh(core_axis_name="c", subcore_axis_name="s")


@functools.partial(
    pl.kernel,
    mesh=_mesh,
    out_type=jax.ShapeDtypeStruct((NW * NQ * S,), jnp.float32),
    compiler_params=pltpu.CompilerParams(needs_layout_passes=False),
    scratch_types=[
        pltpu.VMEM((CHUNK,), jnp.float32),
        pltpu.VMEM((CHUNK,), jnp.float32),
        pltpu.VMEM((CHUNK,), jnp.int32),
        pltpu.VMEM((3, CHUNK), jnp.float32),
    ] + [pltpu.VMEM((S,), jnp.float32) for _ in range(NQ)],
)
def _seg_partials(pred_hbm, targ_hbm, batch_hbm, posf_hbm, out_hbm,
                  pred_v, targ_v, batch_v, pos_v, *accs):
    wid = lax.axis_index("s") * 2 + lax.axis_index("c")
    blk0 = (NBLK * wid) // NW
    blk1 = (NBLK * (wid + 1)) // NW
    nfull = (blk1 - blk0) // CB
    ntail = (blk1 - blk0) - nfull * CB

    zero16 = jnp.zeros((16,), jnp.float32)

    def zero_body(i, carry):
        for a in accs:
            a[pl.ds(i * 16, 16)] = zero16
        return carry

    lax.fori_loop(0, S // 16, zero_body, 0)

    lane = lax.iota(jnp.int32, 16)
    shift = jnp.minimum(lane + 1, 15)
    lane_lt15 = lane < 15
    lane15 = lane == 15

    def do_chunk(base_blk, nblk_chunk):
        npts = nblk_chunk * BLK
        base = pl.multiple_of(base_blk * BLK, BLK)
        pltpu.sync_copy(pred_hbm.at[pl.ds(base, npts)], pred_v.at[pl.ds(0, npts)])
        pltpu.sync_copy(targ_hbm.at[pl.ds(base, npts)], targ_v.at[pl.ds(0, npts)])
        pltpu.sync_copy(batch_hbm.at[pl.ds(base, npts)], batch_v.at[pl.ds(0, npts)])
        pltpu.sync_copy(posf_hbm.at[:, pl.ds(base, npts)], pos_v.at[:, pl.ds(0, npts)])

        def vec_body(i, inner):
            off = i * 16
            b = batch_v[pl.ds(off, 16)]
            p = pred_v[pl.ds(off, 16)]
            t = targ_v[pl.ds(off, 16)]
            px = pos_v[0, pl.ds(off, 16)]
            py = pos_v[1, pl.ds(off, 16)]
            pz = pos_v[2, pl.ds(off, 16)]
            bnext = plsc.load_gather(batch_v, [off + shift])
            m = (b != bnext) & lane_lt15
            m1 = m | lane15
            vals = (p, t, p * t, px * t, py * t, pz * t, px * p, py * p, pz * p)
            for a, vq in zip(accs, vals):
                cs = plsc.cumsum(vq)
                plsc.addupdate_scatter(a, [b], cs, mask=m1)
                plsc.addupdate_scatter(a, [bnext], -cs, mask=m)
            return inner

        lax.fori_loop(0, npts // 16, vec_body, 0)

    def chunk_body(c, carry):
        do_chunk(blk0 + c * CB, CB)
        return carry

    lax.fori_loop(0, nfull, chunk_body, 0)

    def tail_body(j, carry):
        do_chunk(blk0 + nfull * CB + j, 1)
        return carry

    lax.fori_loop(0, ntail, tail_body, 0)

    for q, a in enumerate(accs):
        pltpu.sync_copy(a, out_hbm.at[pl.ds((wid * NQ + q) * S, S)])


def _epilogue_body(parts_ref, out_ref):
    acc = jnp.sum(parts_ref[...], axis=0)
    p_sum = acc[0:1]
    t_sum = acc[1:2]
    inter = acc[2:3]
    dice = (2.0 * inter + 1.0) / (p_sum + t_sum + 1.0)
    dice_loss = jnp.sum(1.0 - dice)
    t_center = acc[3:6] / (t_sum + 1e-10)
    p_center = acc[6:9] / (p_sum + 1e-10)
    diff = t_center - p_center
    pcl = jnp.sqrt(jnp.sum(diff * diff))
    total = dice_loss + pcl
    row_i = lax.broadcasted_iota(jnp.int32, (8, 128), 0)
    col_i = lax.broadcasted_iota(jnp.int32, (8, 128), 1)
    vals = jnp.where(col_i == 0, total,
                     jnp.where(col_i == 1, dice_loss,
                               jnp.where(col_i == 2, pcl, 0.0)))
    out_ref[...] = jnp.where(row_i == 0, vals, 0.0)


def kernel(pred, target, batch, pos):
    pred = pred.reshape(-1).astype(jnp.float32)
    target = target.reshape(-1).astype(jnp.float32)
    posf = pos.astype(jnp.float32).T
    batch = batch.astype(jnp.int32)
    parts = _seg_partials(pred, target, batch, posf).reshape(NW, NQ, S)
    out = pl.pallas_call(
        _epilogue_body,
        out_shape=jax.ShapeDtypeStruct((8, 128), jnp.float32),
    )(parts)
    ce = jnp.zeros((), jnp.float32)
    return (out[0, 0], out[0, 1], out[0, 2], ce)
